# Initial kernel scaffold; baseline (speedup 1.0000x reference)
import jax
import jax.numpy as jnp
from jax.experimental import pallas as pl


def kernel(x, edge_index, batch, W1, b1, W2, b2, FW1, FB1, FW2, FB2):
    raise NotImplementedError("write your pallas kernel here")


# speedup vs baseline: 6.6508x; 6.6508x over previous
import functools

import jax
import jax.numpy as jnp
from jax import lax
from jax.experimental import pallas as pl
from jax.experimental.pallas import tpu as pltpu
from jax.experimental.pallas import tpu_sc as plsc

N = 10000
E = 320000
D_IN = 128
D_H = 256
B = 64
HALF = 128
NC, NS, LANES = 2, 16, 16
CH = 80
SLAB = E // NS
NCHUNK = SLAB // CH
DSLAB = E // (NC * NS)
DNCHUNK = DSLAB // CH
NPAD = 10240
RPT = NPAD // NS
ZR = 128
HN = NPAD // 2
HRPT = HN // NS

_vsc_mesh = functools.partial(
    plsc.VectorSubcoreMesh, core_axis_name="c", subcore_axis_name="s")


@functools.cache
def _deg_kernel():
  @functools.partial(
      pl.kernel,
      mesh=_vsc_mesh(),
      out_type=jax.ShapeDtypeStruct((NC * NPAD, LANES), jnp.float32),
      scratch_types=[
          pltpu.VMEM((DNCHUNK, CH), jnp.int32),
          pltpu.VMEM((CH, LANES), jnp.float32),
          pltpu.VMEM((CH, LANES), jnp.float32),
          pltpu.VMEM_SHARED((HN + 8, LANES), jnp.float32),
      ],
  )
  def k(dst_hbm, out_hbm, dstv, ones, zbuf, acc):
    c = lax.axis_index("c")
    s = lax.axis_index("s")

    @pl.loop(0, CH)
    def _(r):
      zbuf[r, :] = jnp.zeros((LANES,), jnp.float32)
      ones[r, :] = jnp.full((LANES,), 1.0, jnp.float32)

    for p in range(2):
      pltpu.sync_copy(dst_hbm.at[c, s], dstv)

      @pl.loop(0, DNCHUNK)
      def _(kk):
        @pl.loop(0, CH, step=LANES)
        def _(j):
          d = dstv[kk, pl.ds(j, LANES)]
          loc = d - p * HN
          m = (loc >= 0) & (loc < HN)
          dstv[kk, pl.ds(j, LANES)] = jnp.where(m, loc, HN)

      @pl.loop(0, HRPT // CH)
      def _(j):
        pltpu.sync_copy(zbuf, acc.at[pl.ds(s * HRPT + j * CH, CH)])

      plsc.subcore_barrier()

      @pl.loop(0, DNCHUNK)
      def _(kk):
        pltpu.sync_copy(ones, acc.at[dstv.at[kk]], add=True)

      plsc.subcore_barrier()
      pltpu.sync_copy(acc.at[pl.ds(s * HRPT, HRPT)],
                      out_hbm.at[pl.ds(c * NPAD + p * HN + s * HRPT, HRPT)])
      plsc.subcore_barrier()

  return k


@functools.cache
def _segsum_kernel():
  @functools.partial(
      pl.kernel,
      mesh=_vsc_mesh(),
      out_type=jax.ShapeDtypeStruct((NC * NPAD, HALF), jnp.float32),
      scratch_types=[
          pltpu.VMEM((NCHUNK, CH), jnp.int32),
          pltpu.VMEM((NCHUNK, CH), jnp.int32),
          pltpu.VMEM((CH, HALF), jnp.float32),
          pltpu.VMEM_SHARED((NPAD, HALF), jnp.float32),
          pltpu.SemaphoreType.DMA,
      ],
  )
  def k(tab_hbm, src_hbm, dst_hbm, out_hbm, srcv, dstv, rows, acc, sem):
    c = lax.axis_index("c")
    s = lax.axis_index("s")

    @pl.loop(0, CH)
    def _(r):
      @pl.loop(0, HALF, step=LANES)
      def _(j):
        rows[r, pl.ds(j, LANES)] = jnp.zeros((LANES,), jnp.float32)

    pltpu.sync_copy(src_hbm.at[s], srcv)
    pltpu.sync_copy(dst_hbm.at[s], dstv)
    off = c * N

    @pl.loop(0, NCHUNK)
    def _(kk):
      @pl.loop(0, CH, step=LANES)
      def _(j):
        srcv[kk, pl.ds(j, LANES)] = srcv[kk, pl.ds(j, LANES)] + off

    @pl.loop(0, RPT // CH)
    def _(j):
      pltpu.sync_copy(rows, acc.at[pl.ds(s * RPT + j * CH, CH)])

    plsc.subcore_barrier()

    @pl.loop(0, NCHUNK)
    def _(kk):
      pltpu.async_copy(tab_hbm.at[srcv.at[kk]], rows, sem).wait()
      pltpu.sync_copy(rows, acc.at[dstv.at[kk]], add=True)

    plsc.subcore_barrier()
    pltpu.sync_copy(acc.at[pl.ds(s * RPT, RPT)],
                    out_hbm.at[pl.ds(c * NPAD + s * RPT, RPT)])

  return k


@functools.cache
def _segsum_half_kernel():
  @functools.partial(
      pl.kernel,
      mesh=_vsc_mesh(),
      out_type=jax.ShapeDtypeStruct((NC * NPAD, HALF), jnp.float32),
      scratch_types=[
          pltpu.VMEM((NCHUNK, CH), jnp.int32),
          pltpu.VMEM((NCHUNK, CH), jnp.int32),
          pltpu.VMEM((CH,), jnp.int32),
          pltpu.VMEM((CH, HALF), jnp.float32),
          pltpu.VMEM_SHARED((HN + 8, HALF), jnp.float32),
          pltpu.SemaphoreType.DMA,
      ],
  )
  def k(tab_hbm, src_hbm, dst_hbm, out_hbm, srcv, dstv, idxb, rows, acc, sem):
    c = lax.axis_index("c")
    s = lax.axis_index("s")

    pltpu.sync_copy(src_hbm.at[s], srcv)

    for p in range(2):
      @pl.loop(0, CH)
      def _(r):
        @pl.loop(0, HALF, step=LANES)
        def _(j):
          rows[r, pl.ds(j, LANES)] = jnp.zeros((LANES,), jnp.float32)

      pltpu.sync_copy(dst_hbm.at[s], dstv)

      @pl.loop(0, NCHUNK)
      def _(kk):
        @pl.loop(0, CH, step=LANES)
        def _(j):
          d = dstv[kk, pl.ds(j, LANES)]
          loc = d - p * HN
          m = (loc >= 0) & (loc < HN)
          dstv[kk, pl.ds(j, LANES)] = jnp.where(m, loc, HN)

      @pl.loop(0, HRPT // CH)
      def _(j):
        pltpu.sync_copy(rows, acc.at[pl.ds(s * HRPT + j * CH, CH)])

      plsc.subcore_barrier()

      for cc in range(NC):
        @pl.when(c == cc)
        def _():
          @pl.loop(0, NCHUNK)
          def _(kk):
            @pl.loop(0, CH, step=LANES)
            def _(j):
              idxb[pl.ds(j, LANES)] = srcv[kk, pl.ds(j, LANES)] + (cc * N)
            pltpu.sync_copy(tab_hbm.at[idxb], rows)
            pltpu.sync_copy(rows, acc.at[dstv.at[kk]], add=True)

      plsc.subcore_barrier()
      pltpu.sync_copy(acc.at[pl.ds(s * HRPT, HRPT)],
                      out_hbm.at[pl.ds(c * NPAD + p * HN + s * HRPT, HRPT)])
      plsc.subcore_barrier()

  return k


def _tc1_body(x_ref, w_ref, cnt_ref, g_ref, dinv_ref):
  cnt = cnt_ref[...]
  deg = cnt[0, :, :1] + cnt[1, :, :1] + 1.0
  dinv = lax.rsqrt(deg)
  h = jnp.dot(x_ref[...], w_ref[...], preferred_element_type=jnp.float32)
  g = h * dinv
  g_ref[0] = g[:, :HALF]
  g_ref[1] = g[:, HALF:]
  dinv_ref[...] = dinv


def _tc1(x, W1, cnt2):
  R = 1000
  return pl.pallas_call(
      _tc1_body,
      grid=(N // R,),
      in_specs=[
          pl.BlockSpec((R, D_IN), lambda i: (i, 0)),
          pl.BlockSpec((D_IN, D_H), lambda i: (0, 0)),
          pl.BlockSpec((NC, R, LANES), lambda i: (0, i, 0)),
      ],
      out_specs=[
          pl.BlockSpec((NC, R, HALF), lambda i: (0, i, 0)),
          pl.BlockSpec((R, 1), lambda i: (i, 0)),
      ],
      out_shape=[
          jax.ShapeDtypeStruct((NC, N, HALF), jnp.float32),
          jax.ShapeDtypeStruct((N, 1), jnp.float32),
      ],
  )(x, W1, cnt2)


def _tc2_body(acc_ref, g_ref, dinv_ref, b_ref, w_ref, out_ref):
  acc = acc_ref[...]
  g = g_ref[...]
  pre = jnp.concatenate([acc[0] + g[0], acc[1] + g[1]], axis=1)
  dinv = dinv_ref[...]
  t = jnp.maximum(dinv * pre + b_ref[...], 0.0)
  h2 = jnp.dot(t, w_ref[...], preferred_element_type=jnp.float32)
  g2 = h2 * dinv
  out_ref[0] = g2[:, :HALF]
  out_ref[1] = g2[:, HALF:]


def _tc2(acc1, g1, dinv, b1, W2):
  R = 1000
  return pl.pallas_call(
      _tc2_body,
      grid=(N // R,),
      in_specs=[
          pl.BlockSpec((NC, R, HALF), lambda i: (0, i, 0)),
          pl.BlockSpec((NC, R, HALF), lambda i: (0, i, 0)),
          pl.BlockSpec((R, 1), lambda i: (i, 0)),
          pl.BlockSpec((1, D_H), lambda i: (0, 0)),
          pl.BlockSpec((D_H, D_H), lambda i: (0, 0)),
      ],
      out_specs=pl.BlockSpec((NC, R, HALF), lambda i: (0, i, 0)),
      out_shape=jax.ShapeDtypeStruct((NC, N, HALF), jnp.float32),
  )(acc1, g1, dinv, b1, W2)


def _softplus(x):
  return jnp.maximum(x, 0.0) + jnp.log1p(jnp.exp(-jnp.abs(x)))


def _tc3_body(acc_ref, g_ref, dinv_ref, b_ref, batch_ref, fw1_ref, fb1_ref,
              fw2_ref, fb2_ref, rr_ref, ri_ref, zsum, csum):
  i = pl.program_id(0)

  @pl.when(i == 0)
  def _():
    zsum[...] = jnp.zeros_like(zsum)
    csum[...] = jnp.zeros_like(csum)

  acc = acc_ref[...]
  g = g_ref[...]
  pre = jnp.concatenate([acc[0] + g[0], acc[1] + g[1]], axis=1)
  t = jnp.maximum(dinv_ref[...] * pre + b_ref[...], 0.0)
  bb = batch_ref[...][0]
  oh = (lax.broadcasted_iota(jnp.int32, (B, bb.shape[1]), 0) == bb)
  oh = oh.astype(jnp.float32)
  zsum[...] += jnp.dot(oh, t, preferred_element_type=jnp.float32)
  csum[...] += jnp.sum(oh, axis=1, keepdims=True)

  @pl.when(i == pl.num_programs(0) - 1)
  def _():
    z = zsum[...] / jnp.maximum(csum[...], 1.0)
    p1 = jnp.dot(z, fw1_ref[...], preferred_element_type=jnp.float32)
    p1 = jnp.maximum(p1 + fb1_ref[...], 0.0)
    p = jnp.dot(p1, fw2_ref[...], preferred_element_type=jnp.float32)
    p = p + fb2_ref[...]

    def col(j):
      return p[:, j:j + 1]

    d0, d1, d2, d3 = (_softplus(col(j)) for j in range(4))
    r10, i10 = col(4), col(5)
    r20, i20 = col(6), col(7)
    r21, i21 = col(8), col(9)
    r30, i30 = col(10), col(11)
    r31, i31 = col(12), col(13)
    r32, i32 = col(14), col(15)

    m00 = d0 * d0
    m10r, m10i = r10 * d0, i10 * d0
    m11 = r10 * r10 + i10 * i10 + d1 * d1
    m20r, m20i = r20 * d0, i20 * d0
    m21r = r20 * r10 + i20 * i10 + r21 * d1
    m21i = i20 * r10 - r20 * i10 + i21 * d1
    m22 = r20 * r20 + i20 * i20 + r21 * r21 + i21 * i21 + d2 * d2
    m30r, m30i = r30 * d0, i30 * d0
    m31r = r30 * r10 + i30 * i10 + r31 * d1
    m31i = i30 * r10 - r30 * i10 + i31 * d1
    m32r = r30 * r20 + i30 * i20 + r31 * r21 + i31 * i21 + r32 * d2
    m32i = i30 * r20 - r30 * i20 + i31 * r21 - r31 * i21 + i32 * d2
    m33 = (r30 * r30 + i30 * i30 + r31 * r31 + i31 * i31
           + r32 * r32 + i32 * i32 + d3 * d3)
    tr = m00 + m11 + m22 + m33
    zc = jnp.zeros_like(d0)
    rr = jnp.concatenate([
        m00, m10r, m20r, m30r,
        m10r, m11, m21r, m31r,
        m20r, m21r, m22, m32r,
        m30r, m31r, m32r, m33], axis=1) / tr
    ri = jnp.concatenate([
        zc, -m10i, -m20i, -m30i,
        m10i, zc, -m21i, -m31i,
        m20i, m21i, zc, -m32i,
        m30i, m31i, m32i, zc], axis=1) / tr
    rr_ref[...] = rr
    ri_ref[...] = ri


def _tc3(acc2, g2, dinv, b2, batch3, FW1, FB1, FW2, FB2):
  R = 1000
  return pl.pallas_call(
      _tc3_body,
      grid=(N // R,),
      in_specs=[
          pl.BlockSpec((NC, R, HALF), lambda i: (0, i, 0)),
          pl.BlockSpec((NC, R, HALF), lambda i: (0, i, 0)),
          pl.BlockSpec((R, 1), lambda i: (i, 0)),
          pl.BlockSpec((1, D_H), lambda i: (0, 0)),
          pl.BlockSpec((1, 1, R), lambda i: (i, 0, 0)),
          pl.BlockSpec((D_H, 128), lambda i: (0, 0)),
          pl.BlockSpec((1, 128), lambda i: (0, 0)),
          pl.BlockSpec((128, 16), lambda i: (0, 0)),
          pl.BlockSpec((1, 16), lambda i: (0, 0)),
      ],
      out_specs=[
          pl.BlockSpec((B, 16), lambda i: (0, 0)),
          pl.BlockSpec((B, 16), lambda i: (0, 0)),
      ],
      out_shape=[
          jax.ShapeDtypeStruct((B, 16), jnp.float32),
          jax.ShapeDtypeStruct((B, 16), jnp.float32),
      ],
      scratch_shapes=[
          pltpu.VMEM((B, D_H), jnp.float32),
          pltpu.VMEM((B, 1), jnp.float32),
      ],
  )(acc2, g2, dinv, b2, batch3, FW1, FB1, FW2, FB2)


def kernel(x, edge_index, batch, W1, b1, W2, b2, FW1, FB1, FW2, FB2):
  src = edge_index[0]
  dst = edge_index[1]
  dst3d = dst.reshape(NC, NS, DNCHUNK, CH)
  src3 = src.reshape(NS, NCHUNK, CH)
  dst3 = dst.reshape(NS, NCHUNK, CH)

  def _segsum_dbg(table):
    out = [jax.ops.segment_sum(table[c * N + src], dst, num_segments=NPAD)
           for c in range(NC)]
    return jnp.concatenate(out, axis=0)

  cnt = _deg_kernel()(dst3d)
  g1, dinv = _tc1(x, W1, cnt.reshape(NC, NPAD, LANES))
  acc1 = _segsum_half_kernel()(g1.reshape(NC * N, HALF), src3, dst3)
  g2 = _tc2(acc1.reshape(NC, NPAD, HALF), g1, dinv, b1.reshape(1, D_H), W2)
  acc2 = _segsum_half_kernel()(g2.reshape(NC * N, HALF), src3, dst3)
  rr, ri = _tc3(acc2.reshape(NC, NPAD, HALF), g2, dinv, b2.reshape(1, D_H),
                batch.reshape(N // 1000, 1, 1000), FW1, FB1.reshape(1, 128),
                FW2, FB2.reshape(1, 16))
  rho = lax.complex(rr, ri).reshape(B, 4, 4)
  return rho


# speedup vs baseline: 7.4533x; 1.1207x over previous
import functools

import jax
import jax.numpy as jnp
from jax import lax
from jax.experimental import pallas as pl
from jax.experimental.pallas import tpu as pltpu
from jax.experimental.pallas import tpu_sc as plsc

N = 10000
E = 320000
D_IN = 128
D_H = 256
B = 64
HALF = 128
NC, NS, LANES = 2, 16, 16
CH = 80
SLAB = E // NS
NCHUNK = SLAB // CH
DSLAB = E // (NC * NS)
DNCHUNK = DSLAB // CH
NPAD = 10240
RPT = NPAD // NS
ZR = 128
HN = NPAD // 2
HRPT = HN // NS

_vsc_mesh = functools.partial(
    plsc.VectorSubcoreMesh, core_axis_name="c", subcore_axis_name="s")


@functools.cache
def _deg_kernel():
  @functools.partial(
      pl.kernel,
      mesh=_vsc_mesh(),
      out_type=jax.ShapeDtypeStruct((NC * NPAD, LANES), jnp.float32),
      scratch_types=[
          pltpu.VMEM((DNCHUNK, CH), jnp.int32),
          pltpu.VMEM((CH, LANES), jnp.float32),
          pltpu.VMEM((CH, LANES), jnp.float32),
          pltpu.VMEM_SHARED((HN + 8, LANES), jnp.float32),
          pltpu.SemaphoreType.DMA,
      ],
  )
  def k(dst_hbm, out_hbm, dstv, ones, zbuf, acc, sem):
    c = lax.axis_index("c")
    s = lax.axis_index("s")

    @pl.loop(0, CH)
    def _(r):
      zbuf[r, :] = jnp.zeros((LANES,), jnp.float32)
      ones[r, :] = jnp.full((LANES,), 1.0, jnp.float32)

    for p in range(2):
      pltpu.sync_copy(dst_hbm.at[c, s], dstv)

      @pl.loop(0, DNCHUNK)
      def _(kk):
        @pl.loop(0, CH, step=LANES)
        def _(j):
          d = dstv[kk, pl.ds(j, LANES)]
          loc = d - p * HN
          m = (loc >= 0) & (loc < HN)
          dstv[kk, pl.ds(j, LANES)] = jnp.where(m, loc, HN)

      @pl.loop(0, HRPT // CH)
      def _(j):
        pltpu.sync_copy(zbuf, acc.at[pl.ds(s * HRPT + j * CH, CH)])

      plsc.subcore_barrier()

      @pl.loop(0, DNCHUNK)
      def _(kk):
        pltpu.sync_copy(ones, acc.at[dstv.at[kk]], add=True)

      plsc.subcore_barrier()
      pltpu.sync_copy(acc.at[pl.ds(s * HRPT, HRPT)],
                      out_hbm.at[pl.ds(c * NPAD + p * HN + s * HRPT, HRPT)])
      plsc.subcore_barrier()

  return k


@functools.cache
def _segsum_kernel():
  @functools.partial(
      pl.kernel,
      mesh=_vsc_mesh(),
      out_type=jax.ShapeDtypeStruct((NC * NPAD, HALF), jnp.float32),
      scratch_types=[
          pltpu.VMEM((NCHUNK, CH), jnp.int32),
          pltpu.VMEM((NCHUNK, CH), jnp.int32),
          pltpu.VMEM((CH, HALF), jnp.float32),
          pltpu.VMEM_SHARED((NPAD, HALF), jnp.float32),
          pltpu.SemaphoreType.DMA,
      ],
  )
  def k(tab_hbm, src_hbm, dst_hbm, out_hbm, srcv, dstv, rows, acc, sem):
    c = lax.axis_index("c")
    s = lax.axis_index("s")

    @pl.loop(0, CH)
    def _(r):
      @pl.loop(0, HALF, step=LANES)
      def _(j):
        rows[r, pl.ds(j, LANES)] = jnp.zeros((LANES,), jnp.float32)

    pltpu.sync_copy(src_hbm.at[s], srcv)
    pltpu.sync_copy(dst_hbm.at[s], dstv)
    off = c * N

    @pl.loop(0, NCHUNK)
    def _(kk):
      @pl.loop(0, CH, step=LANES)
      def _(j):
        srcv[kk, pl.ds(j, LANES)] = srcv[kk, pl.ds(j, LANES)] + off

    @pl.loop(0, RPT // CH)
    def _(j):
      pltpu.sync_copy(rows, acc.at[pl.ds(s * RPT + j * CH, CH)])

    plsc.subcore_barrier()

    @pl.loop(0, NCHUNK)
    def _(kk):
      pltpu.async_copy(tab_hbm.at[srcv.at[kk]], rows, sem).wait()
      pltpu.sync_copy(rows, acc.at[dstv.at[kk]], add=True)

    plsc.subcore_barrier()
    pltpu.sync_copy(acc.at[pl.ds(s * RPT, RPT)],
                    out_hbm.at[pl.ds(c * NPAD + s * RPT, RPT)])

  return k


@functools.cache
def _segsum_half_kernel():
  @functools.partial(
      pl.kernel,
      mesh=_vsc_mesh(),
      out_type=jax.ShapeDtypeStruct((NC * NPAD, HALF), jnp.float32),
      scratch_types=[
          pltpu.VMEM((NCHUNK, CH), jnp.int32),
          pltpu.VMEM((NCHUNK, CH), jnp.int32),
          pltpu.VMEM((CH,), jnp.int32),
          pltpu.VMEM((CH,), jnp.int32),
          pltpu.VMEM((CH, HALF), jnp.float32),
          pltpu.VMEM((CH, HALF), jnp.float32),
          pltpu.VMEM_SHARED((HN + 8, HALF), jnp.float32),
          pltpu.SemaphoreType.DMA,
          pltpu.SemaphoreType.DMA,
      ],
  )
  def k(tab_hbm, src_hbm, dst_hbm, out_hbm, srcv, dstv, idxb0, idxb1,
        rows0, rows1, acc, gsem0, gsem1):
    c = lax.axis_index("c")
    s = lax.axis_index("s")

    pltpu.sync_copy(src_hbm.at[s], srcv)

    for p in range(2):
      @pl.loop(0, CH)
      def _(r):
        @pl.loop(0, HALF, step=LANES)
        def _(j):
          rows0[r, pl.ds(j, LANES)] = jnp.zeros((LANES,), jnp.float32)

      pltpu.sync_copy(dst_hbm.at[s], dstv)

      @pl.loop(0, NCHUNK)
      def _(kk):
        @pl.loop(0, CH, step=LANES)
        def _(j):
          d = dstv[kk, pl.ds(j, LANES)]
          loc = d - p * HN
          m = (loc >= 0) & (loc < HN)
          dstv[kk, pl.ds(j, LANES)] = jnp.where(m, loc, HN)

      @pl.loop(0, HRPT // CH)
      def _(j):
        pltpu.sync_copy(rows0, acc.at[pl.ds(s * HRPT + j * CH, CH)])

      plsc.subcore_barrier()

      for cc in range(NC):
        @pl.when(c == cc)
        def _():
          def build_idx(idxb, kk):
            @pl.loop(0, CH, step=LANES)
            def _(j):
              idxb[pl.ds(j, LANES)] = srcv[kk, pl.ds(j, LANES)] + (cc * N)

          @pl.loop(0, NCHUNK // 2)
          def _(kk2):
            k0 = kk2 * 2
            build_idx(idxb0, k0)
            pltpu.async_copy(tab_hbm.at[idxb0], rows0, gsem0).wait()
            s0 = pltpu.async_copy(rows0, acc.at[dstv.at[k0]], gsem1, add=True)
            build_idx(idxb1, k0 + 1)
            pltpu.async_copy(tab_hbm.at[idxb1], rows1, gsem0).wait()
            s0.wait()
            pltpu.sync_copy(rows1, acc.at[dstv.at[k0 + 1]], add=True)

      plsc.subcore_barrier()
      pltpu.sync_copy(acc.at[pl.ds(s * HRPT, HRPT)],
                      out_hbm.at[pl.ds(c * NPAD + p * HN + s * HRPT, HRPT)])
      plsc.subcore_barrier()

  return k


def _tc1_body(x_ref, w_ref, cnt_ref, g_ref, dinv_ref):
  cnt = cnt_ref[...]
  deg = cnt[0, :, :1] + cnt[1, :, :1] + 1.0
  dinv = lax.rsqrt(deg)
  h = jnp.dot(x_ref[...], w_ref[...], preferred_element_type=jnp.float32)
  g = h * dinv
  g_ref[0] = g[:, :HALF]
  g_ref[1] = g[:, HALF:]
  dinv_ref[...] = dinv


def _tc1(x, W1, cnt2):
  R = 1000
  return pl.pallas_call(
      _tc1_body,
      grid=(N // R,),
      in_specs=[
          pl.BlockSpec((R, D_IN), lambda i: (i, 0)),
          pl.BlockSpec((D_IN, D_H), lambda i: (0, 0)),
          pl.BlockSpec((NC, R, LANES), lambda i: (0, i, 0)),
      ],
      out_specs=[
          pl.BlockSpec((NC, R, HALF), lambda i: (0, i, 0)),
          pl.BlockSpec((R, 1), lambda i: (i, 0)),
      ],
      out_shape=[
          jax.ShapeDtypeStruct((NC, N, HALF), jnp.float32),
          jax.ShapeDtypeStruct((N, 1), jnp.float32),
      ],
  )(x, W1, cnt2)


def _tc2_body(acc_ref, g_ref, dinv_ref, b_ref, w_ref, out_ref):
  acc = acc_ref[...]
  g = g_ref[...]
  pre = jnp.concatenate([acc[0] + g[0], acc[1] + g[1]], axis=1)
  dinv = dinv_ref[...]
  t = jnp.maximum(dinv * pre + b_ref[...], 0.0)
  h2 = jnp.dot(t, w_ref[...], preferred_element_type=jnp.float32)
  g2 = h2 * dinv
  out_ref[0] = g2[:, :HALF]
  out_ref[1] = g2[:, HALF:]


def _tc2(acc1, g1, dinv, b1, W2):
  R = 1000
  return pl.pallas_call(
      _tc2_body,
      grid=(N // R,),
      in_specs=[
          pl.BlockSpec((NC, R, HALF), lambda i: (0, i, 0)),
          pl.BlockSpec((NC, R, HALF), lambda i: (0, i, 0)),
          pl.BlockSpec((R, 1), lambda i: (i, 0)),
          pl.BlockSpec((1, D_H), lambda i: (0, 0)),
          pl.BlockSpec((D_H, D_H), lambda i: (0, 0)),
      ],
      out_specs=pl.BlockSpec((NC, R, HALF), lambda i: (0, i, 0)),
      out_shape=jax.ShapeDtypeStruct((NC, N, HALF), jnp.float32),
  )(acc1, g1, dinv, b1, W2)


def _softplus(x):
  return jnp.maximum(x, 0.0) + jnp.log1p(jnp.exp(-jnp.abs(x)))


def _tc3_body(acc_ref, g_ref, dinv_ref, b_ref, batch_ref, fw1_ref, fb1_ref,
              fw2_ref, fb2_ref, rr_ref, ri_ref, zsum, csum):
  i = pl.program_id(0)

  @pl.when(i == 0)
  def _():
    zsum[...] = jnp.zeros_like(zsum)
    csum[...] = jnp.zeros_like(csum)

  acc = acc_ref[...]
  g = g_ref[...]
  pre = jnp.concatenate([acc[0] + g[0], acc[1] + g[1]], axis=1)
  t = jnp.maximum(dinv_ref[...] * pre + b_ref[...], 0.0)
  bb = batch_ref[...][0]
  oh = (lax.broadcasted_iota(jnp.int32, (B, bb.shape[1]), 0) == bb)
  oh = oh.astype(jnp.float32)
  zsum[...] += jnp.dot(oh, t, preferred_element_type=jnp.float32)
  csum[...] += jnp.sum(oh, axis=1, keepdims=True)

  @pl.when(i == pl.num_programs(0) - 1)
  def _():
    z = zsum[...] / jnp.maximum(csum[...], 1.0)
    p1 = jnp.dot(z, fw1_ref[...], preferred_element_type=jnp.float32)
    p1 = jnp.maximum(p1 + fb1_ref[...], 0.0)
    p = jnp.dot(p1, fw2_ref[...], preferred_element_type=jnp.float32)
    p = p + fb2_ref[...]

    def col(j):
      return p[:, j:j + 1]

    d0, d1, d2, d3 = (_softplus(col(j)) for j in range(4))
    r10, i10 = col(4), col(5)
    r20, i20 = col(6), col(7)
    r21, i21 = col(8), col(9)
    r30, i30 = col(10), col(11)
    r31, i31 = col(12), col(13)
    r32, i32 = col(14), col(15)

    m00 = d0 * d0
    m10r, m10i = r10 * d0, i10 * d0
    m11 = r10 * r10 + i10 * i10 + d1 * d1
    m20r, m20i = r20 * d0, i20 * d0
    m21r = r20 * r10 + i20 * i10 + r21 * d1
    m21i = i20 * r10 - r20 * i10 + i21 * d1
    m22 = r20 * r20 + i20 * i20 + r21 * r21 + i21 * i21 + d2 * d2
    m30r, m30i = r30 * d0, i30 * d0
    m31r = r30 * r10 + i30 * i10 + r31 * d1
    m31i = i30 * r10 - r30 * i10 + i31 * d1
    m32r = r30 * r20 + i30 * i20 + r31 * r21 + i31 * i21 + r32 * d2
    m32i = i30 * r20 - r30 * i20 + i31 * r21 - r31 * i21 + i32 * d2
    m33 = (r30 * r30 + i30 * i30 + r31 * r31 + i31 * i31
           + r32 * r32 + i32 * i32 + d3 * d3)
    tr = m00 + m11 + m22 + m33
    zc = jnp.zeros_like(d0)
    rr = jnp.concatenate([
        m00, m10r, m20r, m30r,
        m10r, m11, m21r, m31r,
        m20r, m21r, m22, m32r,
        m30r, m31r, m32r, m33], axis=1) / tr
    ri = jnp.concatenate([
        zc, -m10i, -m20i, -m30i,
        m10i, zc, -m21i, -m31i,
        m20i, m21i, zc, -m32i,
        m30i, m31i, m32i, zc], axis=1) / tr
    rr_ref[...] = rr
    ri_ref[...] = ri


def _tc3(acc2, g2, dinv, b2, batch3, FW1, FB1, FW2, FB2):
  R = 1000
  return pl.pallas_call(
      _tc3_body,
      grid=(N // R,),
      in_specs=[
          pl.BlockSpec((NC, R, HALF), lambda i: (0, i, 0)),
          pl.BlockSpec((NC, R, HALF), lambda i: (0, i, 0)),
          pl.BlockSpec((R, 1), lambda i: (i, 0)),
          pl.BlockSpec((1, D_H), lambda i: (0, 0)),
          pl.BlockSpec((1, 1, R), lambda i: (i, 0, 0)),
          pl.BlockSpec((D_H, 128), lambda i: (0, 0)),
          pl.BlockSpec((1, 128), lambda i: (0, 0)),
          pl.BlockSpec((128, 16), lambda i: (0, 0)),
          pl.BlockSpec((1, 16), lambda i: (0, 0)),
      ],
      out_specs=[
          pl.BlockSpec((B, 16), lambda i: (0, 0)),
          pl.BlockSpec((B, 16), lambda i: (0, 0)),
      ],
      out_shape=[
          jax.ShapeDtypeStruct((B, 16), jnp.float32),
          jax.ShapeDtypeStruct((B, 16), jnp.float32),
      ],
      scratch_shapes=[
          pltpu.VMEM((B, D_H), jnp.float32),
          pltpu.VMEM((B, 1), jnp.float32),
      ],
  )(acc2, g2, dinv, b2, batch3, FW1, FB1, FW2, FB2)


def kernel(x, edge_index, batch, W1, b1, W2, b2, FW1, FB1, FW2, FB2):
  src = edge_index[0]
  dst = edge_index[1]
  dst3d = dst.reshape(NC, NS, DNCHUNK, CH)
  src3 = src.reshape(NS, NCHUNK, CH)
  dst3 = dst.reshape(NS, NCHUNK, CH)

  def _segsum_dbg(table):
    out = [jax.ops.segment_sum(table[c * N + src], dst, num_segments=NPAD)
           for c in range(NC)]
    return jnp.concatenate(out, axis=0)

  cnt = _deg_kernel()(dst3d)
  g1, dinv = _tc1(x, W1, cnt.reshape(NC, NPAD, LANES))
  acc1 = _segsum_half_kernel()(g1.reshape(NC * N, HALF), src3, dst3)
  g2 = _tc2(acc1.reshape(NC, NPAD, HALF), g1, dinv, b1.reshape(1, D_H), W2)
  acc2 = _segsum_half_kernel()(g2.reshape(NC * N, HALF), src3, dst3)
  rr, ri = _tc3(acc2.reshape(NC, NPAD, HALF), g2, dinv, b2.reshape(1, D_H),
                batch.reshape(N // 1000, 1, 1000), FW1, FB1.reshape(1, 128),
                FW2, FB2.reshape(1, 16))
  rho = lax.complex(rr, ri).reshape(B, 4, 4)
  return rho


# speedup vs baseline: 8.3252x; 1.1170x over previous
import functools

import jax
import jax.numpy as jnp
from jax import lax
from jax.experimental import pallas as pl
from jax.experimental.pallas import tpu as pltpu
from jax.experimental.pallas import tpu_sc as plsc

N = 10000
E = 320000
D_IN = 128
D_H = 256
B = 64
HALF = 128
NC, NS, LANES = 2, 16, 16
CH = 80
SLAB = E // NS
NCHUNK = SLAB // CH
DSLAB = E // (NC * NS)
DNCHUNK = DSLAB // CH
NPAD = 10240
RPT = NPAD // NS
ZR = 128
HN = NPAD // 2
HRPT = HN // NS

_vsc_mesh = functools.partial(
    plsc.VectorSubcoreMesh, core_axis_name="c", subcore_axis_name="s")


@functools.cache
def _deg_kernel():
  @functools.partial(
      pl.kernel,
      mesh=_vsc_mesh(),
      out_type=jax.ShapeDtypeStruct((NC * NPAD, LANES), jnp.float32),
      scratch_types=[
          pltpu.VMEM((DNCHUNK, CH), jnp.int32),
          pltpu.VMEM((CH, LANES), jnp.float32),
          pltpu.VMEM((CH, LANES), jnp.float32),
          pltpu.VMEM_SHARED((NPAD, LANES), jnp.float32),
          pltpu.SemaphoreType.DMA,
      ],
  )
  def k(dst_hbm, out_hbm, dstv, ones, zbuf, acc, sem):
    c = lax.axis_index("c")
    s = lax.axis_index("s")

    @pl.loop(0, CH)
    def _(r):
      zbuf[r, :] = jnp.zeros((LANES,), jnp.float32)
      ones[r, :] = jnp.full((LANES,), 1.0, jnp.float32)

    pltpu.sync_copy(dst_hbm.at[c, s], dstv)

    @pl.loop(0, RPT // CH)
    def _(j):
      pltpu.sync_copy(zbuf, acc.at[pl.ds(s * RPT + j * CH, CH)])

    plsc.subcore_barrier()

    @pl.loop(0, DNCHUNK)
    def _(kk):
      pltpu.sync_copy(ones, acc.at[dstv.at[kk]], add=True)

    plsc.subcore_barrier()
    pltpu.sync_copy(acc.at[pl.ds(s * RPT, RPT)],
                    out_hbm.at[pl.ds(c * NPAD + s * RPT, RPT)])

  return k


@functools.cache
def _segsum_kernel():
  @functools.partial(
      pl.kernel,
      mesh=_vsc_mesh(),
      out_type=jax.ShapeDtypeStruct((NC * NPAD, HALF), jnp.float32),
      scratch_types=[
          pltpu.VMEM((NCHUNK, CH), jnp.int32),
          pltpu.VMEM((NCHUNK, CH), jnp.int32),
          pltpu.VMEM((CH, HALF), jnp.float32),
          pltpu.VMEM_SHARED((NPAD, HALF), jnp.float32),
          pltpu.SemaphoreType.DMA,
      ],
  )
  def k(tab_hbm, src_hbm, dst_hbm, out_hbm, srcv, dstv, rows, acc, sem):
    c = lax.axis_index("c")
    s = lax.axis_index("s")

    @pl.loop(0, CH)
    def _(r):
      @pl.loop(0, HALF, step=LANES)
      def _(j):
        rows[r, pl.ds(j, LANES)] = jnp.zeros((LANES,), jnp.float32)

    pltpu.sync_copy(src_hbm.at[s], srcv)
    pltpu.sync_copy(dst_hbm.at[s], dstv)
    off = c * N

    @pl.loop(0, NCHUNK)
    def _(kk):
      @pl.loop(0, CH, step=LANES)
      def _(j):
        srcv[kk, pl.ds(j, LANES)] = srcv[kk, pl.ds(j, LANES)] + off

    @pl.loop(0, RPT // CH)
    def _(j):
      pltpu.sync_copy(rows, acc.at[pl.ds(s * RPT + j * CH, CH)])

    plsc.subcore_barrier()

    @pl.loop(0, NCHUNK)
    def _(kk):
      pltpu.async_copy(tab_hbm.at[srcv.at[kk]], rows, sem).wait()
      pltpu.sync_copy(rows, acc.at[dstv.at[kk]], add=True)

    plsc.subcore_barrier()
    pltpu.sync_copy(acc.at[pl.ds(s * RPT, RPT)],
                    out_hbm.at[pl.ds(c * NPAD + s * RPT, RPT)])

  return k


@functools.cache
def _segsum_half_kernel():
  @functools.partial(
      pl.kernel,
      mesh=_vsc_mesh(),
      out_type=jax.ShapeDtypeStruct((NC * NPAD, HALF), jnp.float32),
      scratch_types=[
          pltpu.VMEM((NCHUNK, CH), jnp.int32),
          pltpu.VMEM((NCHUNK, CH), jnp.int32),
          pltpu.VMEM((CH,), jnp.int32),
          pltpu.VMEM((CH,), jnp.int32),
          pltpu.VMEM((CH, HALF), jnp.float32),
          pltpu.VMEM((CH, HALF), jnp.float32),
          pltpu.VMEM_SHARED((HN + 8, HALF), jnp.float32),
          pltpu.SemaphoreType.DMA,
          pltpu.SemaphoreType.DMA,
      ],
  )
  def k(tab_hbm, src_hbm, dst_hbm, out_hbm, srcv, dstv, idxb0, idxb1,
        rows0, rows1, acc, gsem0, gsem1):
    c = lax.axis_index("c")
    s = lax.axis_index("s")

    pltpu.sync_copy(src_hbm.at[s], srcv)

    for p in range(2):
      @pl.loop(0, CH)
      def _(r):
        @pl.loop(0, HALF, step=LANES)
        def _(j):
          rows0[r, pl.ds(j, LANES)] = jnp.zeros((LANES,), jnp.float32)

      pltpu.sync_copy(dst_hbm.at[s], dstv)

      @pl.loop(0, NCHUNK)
      def _(kk):
        @pl.loop(0, CH, step=LANES)
        def _(j):
          d = dstv[kk, pl.ds(j, LANES)]
          loc = d - p * HN
          m = (loc >= 0) & (loc < HN)
          dstv[kk, pl.ds(j, LANES)] = jnp.where(m, loc, HN)

      @pl.loop(0, HRPT // CH)
      def _(j):
        pltpu.sync_copy(rows0, acc.at[pl.ds(s * HRPT + j * CH, CH)])

      plsc.subcore_barrier()

      for cc in range(NC):
        @pl.when(c == cc)
        def _():
          def build_idx(idxb, kk):
            @pl.loop(0, CH, step=LANES)
            def _(j):
              idxb[pl.ds(j, LANES)] = srcv[kk, pl.ds(j, LANES)] + (cc * N)

          @pl.loop(0, NCHUNK // 2)
          def _(kk2):
            k0 = kk2 * 2
            build_idx(idxb0, k0)
            pltpu.async_copy(tab_hbm.at[idxb0], rows0, gsem0).wait()
            s0 = pltpu.async_copy(rows0, acc.at[dstv.at[k0]], gsem1, add=True)
            build_idx(idxb1, k0 + 1)
            pltpu.async_copy(tab_hbm.at[idxb1], rows1, gsem0).wait()
            s0.wait()
            pltpu.sync_copy(rows1, acc.at[dstv.at[k0 + 1]], add=True)

      plsc.subcore_barrier()
      pltpu.sync_copy(acc.at[pl.ds(s * HRPT, HRPT)],
                      out_hbm.at[pl.ds(c * NPAD + p * HN + s * HRPT, HRPT)])
      plsc.subcore_barrier()

  return k


def _tc1_body(x_ref, w_ref, cnt_ref, g_ref, dinv_ref):
  cnt = cnt_ref[...]
  deg = cnt[0, :, :1] + cnt[1, :, :1] + 1.0
  dinv = lax.rsqrt(deg)
  h = jnp.dot(x_ref[...], w_ref[...], preferred_element_type=jnp.float32)
  g = h * dinv
  g_ref[0] = g[:, :HALF]
  g_ref[1] = g[:, HALF:]
  dinv_ref[...] = dinv


def _tc1(x, W1, cnt2):
  R = 1000
  return pl.pallas_call(
      _tc1_body,
      grid=(N // R,),
      in_specs=[
          pl.BlockSpec((R, D_IN), lambda i: (i, 0)),
          pl.BlockSpec((D_IN, D_H), lambda i: (0, 0)),
          pl.BlockSpec((NC, R, LANES), lambda i: (0, i, 0)),
      ],
      out_specs=[
          pl.BlockSpec((NC, R, HALF), lambda i: (0, i, 0)),
          pl.BlockSpec((R, 1), lambda i: (i, 0)),
      ],
      out_shape=[
          jax.ShapeDtypeStruct((NC, N, HALF), jnp.float32),
          jax.ShapeDtypeStruct((N, 1), jnp.float32),
      ],
  )(x, W1, cnt2)


def _tc2_body(acc_ref, g_ref, dinv_ref, b_ref, w_ref, out_ref):
  acc = acc_ref[...]
  g = g_ref[...]
  pre = jnp.concatenate([acc[0] + g[0], acc[1] + g[1]], axis=1)
  dinv = dinv_ref[...]
  t = jnp.maximum(dinv * pre + b_ref[...], 0.0)
  h2 = jnp.dot(t, w_ref[...], preferred_element_type=jnp.float32)
  g2 = h2 * dinv
  out_ref[0] = g2[:, :HALF]
  out_ref[1] = g2[:, HALF:]


def _tc2(acc1, g1, dinv, b1, W2):
  R = 1000
  return pl.pallas_call(
      _tc2_body,
      grid=(N // R,),
      in_specs=[
          pl.BlockSpec((NC, R, HALF), lambda i: (0, i, 0)),
          pl.BlockSpec((NC, R, HALF), lambda i: (0, i, 0)),
          pl.BlockSpec((R, 1), lambda i: (i, 0)),
          pl.BlockSpec((1, D_H), lambda i: (0, 0)),
          pl.BlockSpec((D_H, D_H), lambda i: (0, 0)),
      ],
      out_specs=pl.BlockSpec((NC, R, HALF), lambda i: (0, i, 0)),
      out_shape=jax.ShapeDtypeStruct((NC, N, HALF), jnp.float32),
  )(acc1, g1, dinv, b1, W2)


def _softplus(x):
  return jnp.maximum(x, 0.0) + jnp.log1p(jnp.exp(-jnp.abs(x)))


def _tc3_body(acc_ref, g_ref, dinv_ref, b_ref, batch_ref, fw1_ref, fb1_ref,
              fw2_ref, fb2_ref, rr_ref, ri_ref, zsum, csum):
  i = pl.program_id(0)

  @pl.when(i == 0)
  def _():
    zsum[...] = jnp.zeros_like(zsum)
    csum[...] = jnp.zeros_like(csum)

  acc = acc_ref[...]
  g = g_ref[...]
  pre = jnp.concatenate([acc[0] + g[0], acc[1] + g[1]], axis=1)
  t = jnp.maximum(dinv_ref[...] * pre + b_ref[...], 0.0)
  bb = batch_ref[...][0]
  oh = (lax.broadcasted_iota(jnp.int32, (B, bb.shape[1]), 0) == bb)
  oh = oh.astype(jnp.float32)
  zsum[...] += jnp.dot(oh, t, preferred_element_type=jnp.float32)
  csum[...] += jnp.sum(oh, axis=1, keepdims=True)

  @pl.when(i == pl.num_programs(0) - 1)
  def _():
    z = zsum[...] / jnp.maximum(csum[...], 1.0)
    p1 = jnp.dot(z, fw1_ref[...], preferred_element_type=jnp.float32)
    p1 = jnp.maximum(p1 + fb1_ref[...], 0.0)
    p = jnp.dot(p1, fw2_ref[...], preferred_element_type=jnp.float32)
    p = p + fb2_ref[...]

    def col(j):
      return p[:, j:j + 1]

    d0, d1, d2, d3 = (_softplus(col(j)) for j in range(4))
    r10, i10 = col(4), col(5)
    r20, i20 = col(6), col(7)
    r21, i21 = col(8), col(9)
    r30, i30 = col(10), col(11)
    r31, i31 = col(12), col(13)
    r32, i32 = col(14), col(15)

    m00 = d0 * d0
    m10r, m10i = r10 * d0, i10 * d0
    m11 = r10 * r10 + i10 * i10 + d1 * d1
    m20r, m20i = r20 * d0, i20 * d0
    m21r = r20 * r10 + i20 * i10 + r21 * d1
    m21i = i20 * r10 - r20 * i10 + i21 * d1
    m22 = r20 * r20 + i20 * i20 + r21 * r21 + i21 * i21 + d2 * d2
    m30r, m30i = r30 * d0, i30 * d0
    m31r = r30 * r10 + i30 * i10 + r31 * d1
    m31i = i30 * r10 - r30 * i10 + i31 * d1
    m32r = r30 * r20 + i30 * i20 + r31 * r21 + i31 * i21 + r32 * d2
    m32i = i30 * r20 - r30 * i20 + i31 * r21 - r31 * i21 + i32 * d2
    m33 = (r30 * r30 + i30 * i30 + r31 * r31 + i31 * i31
           + r32 * r32 + i32 * i32 + d3 * d3)
    tr = m00 + m11 + m22 + m33
    zc = jnp.zeros_like(d0)
    rr = jnp.concatenate([
        m00, m10r, m20r, m30r,
        m10r, m11, m21r, m31r,
        m20r, m21r, m22, m32r,
        m30r, m31r, m32r, m33], axis=1) / tr
    ri = jnp.concatenate([
        zc, -m10i, -m20i, -m30i,
        m10i, zc, -m21i, -m31i,
        m20i, m21i, zc, -m32i,
        m30i, m31i, m32i, zc], axis=1) / tr
    rr_ref[...] = rr
    ri_ref[...] = ri


def _tc3(acc2, g2, dinv, b2, batch3, FW1, FB1, FW2, FB2):
  R = 1000
  return pl.pallas_call(
      _tc3_body,
      grid=(N // R,),
      in_specs=[
          pl.BlockSpec((NC, R, HALF), lambda i: (0, i, 0)),
          pl.BlockSpec((NC, R, HALF), lambda i: (0, i, 0)),
          pl.BlockSpec((R, 1), lambda i: (i, 0)),
          pl.BlockSpec((1, D_H), lambda i: (0, 0)),
          pl.BlockSpec((1, 1, R), lambda i: (i, 0, 0)),
          pl.BlockSpec((D_H, 128), lambda i: (0, 0)),
          pl.BlockSpec((1, 128), lambda i: (0, 0)),
          pl.BlockSpec((128, 16), lambda i: (0, 0)),
          pl.BlockSpec((1, 16), lambda i: (0, 0)),
      ],
      out_specs=[
          pl.BlockSpec((B, 16), lambda i: (0, 0)),
          pl.BlockSpec((B, 16), lambda i: (0, 0)),
      ],
      out_shape=[
          jax.ShapeDtypeStruct((B, 16), jnp.float32),
          jax.ShapeDtypeStruct((B, 16), jnp.float32),
      ],
      scratch_shapes=[
          pltpu.VMEM((B, D_H), jnp.float32),
          pltpu.VMEM((B, 1), jnp.float32),
      ],
  )(acc2, g2, dinv, b2, batch3, FW1, FB1, FW2, FB2)


def kernel(x, edge_index, batch, W1, b1, W2, b2, FW1, FB1, FW2, FB2):
  src = edge_index[0]
  dst = edge_index[1]
  dst3d = dst.reshape(NC, NS, DNCHUNK, CH)
  src3 = src.reshape(NS, NCHUNK, CH)
  dst3 = dst.reshape(NS, NCHUNK, CH)

  def _segsum_dbg(table):
    out = [jax.ops.segment_sum(table[c * N + src], dst, num_segments=NPAD)
           for c in range(NC)]
    return jnp.concatenate(out, axis=0)

  cnt = _deg_kernel()(dst3d)
  g1, dinv = _tc1(x, W1, cnt.reshape(NC, NPAD, LANES))
  acc1 = _segsum_half_kernel()(g1.reshape(NC * N, HALF), src3, dst3)
  g2 = _tc2(acc1.reshape(NC, NPAD, HALF), g1, dinv, b1.reshape(1, D_H), W2)
  acc2 = _segsum_half_kernel()(g2.reshape(NC * N, HALF), src3, dst3)
  rr, ri = _tc3(acc2.reshape(NC, NPAD, HALF), g2, dinv, b2.reshape(1, D_H),
                batch.reshape(N // 1000, 1, 1000), FW1, FB1.reshape(1, 128),
                FW2, FB2.reshape(1, 16))
  rho = lax.complex(rr, ri).reshape(B, 4, 4)
  return rho
